# baseline (speedup 1.0000x reference)
import functools
import math

import jax
import jax.numpy as jnp
from jax import lax
from jax.experimental import pallas as pl
from jax.experimental.pallas import tpu as pltpu
from jax.experimental.pallas import tpu_sc as plsc

N_NODES = 10000
D = 128
EPS = 1e-5
ISQ = 1.0 / math.sqrt(1.0 + EPS)

NC = 2
NS = 16
NW = NC * NS
CHUNK = 128
N_PAD = 10240

ROWS_PER_SUB = N_PAD // NS


def _sc_mesh():
    return plsc.VectorSubcoreMesh(
        core_axis_name="c", subcore_axis_name="s", num_cores=NC, num_subcores=NS
    )


def _make_deg_kernel(chunks_per_tile):
    @functools.partial(
        pl.kernel,
        mesh=_sc_mesh(),
        out_type=jax.ShapeDtypeStruct((NC, N_PAD, D), jnp.float32),
        scratch_types=[
            pltpu.VMEM((chunks_per_tile, CHUNK), jnp.int32),
            pltpu.VMEM((CHUNK, D), jnp.float32),
            pltpu.VMEM_SHARED((N_PAD, D), jnp.float32),
            pltpu.SemaphoreType.DMA,
        ],
    )
    def deg_kernel(dst_hbm, ones_hbm, z16_hbm, out_hbm, dst_v, ones_v, acc, sem):
        cid = lax.axis_index("c")
        sid = lax.axis_index("s")
        wid = sid * NC + cid
        pltpu.sync_copy(
            z16_hbm.at[pl.ds(sid * ROWS_PER_SUB, ROWS_PER_SUB)],
            acc.at[pl.ds(sid * ROWS_PER_SUB, ROWS_PER_SUB)],
        )
        pltpu.sync_copy(ones_hbm, ones_v)
        pltpu.sync_copy(
            dst_hbm.at[pl.ds(wid * chunks_per_tile, chunks_per_tile)], dst_v)
        plsc.subcore_barrier()

        def fire(j, carry):
            pltpu.async_copy(ones_v, acc.at[dst_v.at[j]], sem, add=True)
            return carry

        lax.fori_loop(0, chunks_per_tile, fire, 0)

        def drain(j, carry):
            pltpu.make_async_copy(ones_v, acc.at[dst_v.at[j]], sem).wait()
            return carry

        lax.fori_loop(0, chunks_per_tile, drain, 0)
        plsc.subcore_barrier()
        pltpu.sync_copy(
            acc.at[pl.ds(sid * ROWS_PER_SUB, ROWS_PER_SUB)],
            out_hbm.at[cid, pl.ds(sid * ROWS_PER_SUB, ROWS_PER_SUB)],
        )

    return deg_kernel


AGG_Q = 32


def _make_agg_kernel(n_chunks):
    assert n_chunks % AGG_Q == 0

    @functools.partial(
        pl.kernel,
        mesh=_sc_mesh(),
        out_type=jax.ShapeDtypeStruct((N_PAD, D), jnp.float32),
        scratch_types=[
            pltpu.VMEM((AGG_Q, CHUNK), jnp.int32),
            pltpu.VMEM((AGG_Q, CHUNK), jnp.int32),
            pltpu.VMEM((2 * CHUNK, D), jnp.float32),
            pltpu.VMEM_SHARED((N_PAD, D), jnp.float32),
            pltpu.SemaphoreType.DMA,
            pltpu.SemaphoreType.DMA,
        ],
    )
    def agg_kernel(hs_hbm, src_hbm, dst_hbm, z_hbm, out_hbm,
                   src_v, dst_v, rows_v, acc, gsem, ssem):
        cid = lax.axis_index("c")
        sid = lax.axis_index("s")

        @pl.when(cid == 0)
        def _():
            pltpu.sync_copy(
                z_hbm.at[pl.ds(sid * ROWS_PER_SUB, ROWS_PER_SUB)],
                acc.at[pl.ds(sid * ROWS_PER_SUB, ROWS_PER_SUB)],
            )
            plsc.subcore_barrier()

            q = AGG_Q
            for phase in range(n_chunks // q):
                base = sid * n_chunks + phase * q
                pltpu.sync_copy(src_hbm.at[pl.ds(base, q)], src_v)
                pltpu.sync_copy(dst_hbm.at[pl.ds(base, q)], dst_v)

                pltpu.async_copy(hs_hbm.at[src_v.at[0]],
                                 rows_v.at[pl.ds(0, CHUNK)], gsem)
                pltpu.async_copy(hs_hbm.at[src_v.at[1]],
                                 rows_v.at[pl.ds(CHUNK, CHUNK)], gsem)

                def body(j, carry):
                    b = lax.rem(j, 2) * CHUNK
                    bslc = rows_v.at[pl.ds(b, CHUNK)]
                    pltpu.make_async_copy(hs_hbm.at[src_v.at[j]], bslc,
                                          gsem).wait()
                    sdesc = pltpu.async_copy(bslc, acc.at[dst_v.at[j]], ssem,
                                             add=True)
                    sdesc.wait()

                    @pl.when(j + 2 < q)
                    def _():
                        pltpu.async_copy(hs_hbm.at[src_v.at[j + 2]], bslc, gsem)

                    return carry

                lax.fori_loop(0, q, body, 0)

            plsc.subcore_barrier()
            pltpu.sync_copy(
                acc.at[pl.ds(sid * ROWS_PER_SUB, ROWS_PER_SUB)],
                out_hbm.at[pl.ds(sid * ROWS_PER_SUB, ROWS_PER_SUB)],
            )

    return agg_kernel


BR = 1024


def _dis_of(degp_ref):
    return lax.rsqrt(degp_ref[0, :, 0:1] + degp_ref[1, :, 0:1] + 1.0)


def _tc_first_body(x_ref, degp_ref, w_ref, o_ref):
    dis = _dis_of(degp_ref)
    h = jnp.dot(x_ref[...], w_ref[...], preferred_element_type=jnp.float32)
    o_ref[...] = h * dis


def _tc_mid_body(agg_ref, hs_ref, degp_ref, w_ref, b_ref, g_ref, be_ref, o_ref):
    dis = _dis_of(degp_ref)
    conv = (agg_ref[...] + hs_ref[...]) * dis + b_ref[...]
    u = jnp.maximum(conv * (g_ref[...] * ISQ) + be_ref[...], 0.0)
    o_ref[...] = jnp.dot(u, w_ref[...], preferred_element_type=jnp.float32) * dis


def _tc_final_body(agg_ref, hs_ref, degp_ref, b_ref, g_ref, be_ref, o_ref):
    dis = _dis_of(degp_ref)
    conv = (agg_ref[...] + hs_ref[...]) * dis + b_ref[...]
    o_ref[...] = conv * (g_ref[...] * ISQ) + be_ref[...]


_GRID = (N_PAD // BR,)
_SPEC_ROWS = pl.BlockSpec((BR, D), lambda i: (i, 0))
_SPEC_DEGP = pl.BlockSpec((2, BR, D), lambda i: (0, i, 0))
_SPEC_W = pl.BlockSpec((D, D), lambda i: (0, 0))
_SPEC_VEC = pl.BlockSpec((1, D), lambda i: (0, 0))
_OUT_ROWS = jax.ShapeDtypeStruct((N_PAD, D), jnp.float32)


def _tc_first(x_pad, degp, W):
    return pl.pallas_call(
        _tc_first_body,
        grid=_GRID,
        in_specs=[_SPEC_ROWS, _SPEC_DEGP, _SPEC_W],
        out_specs=_SPEC_ROWS,
        out_shape=_OUT_ROWS,
    )(x_pad, degp, W)


def _tc_mid(aggp, hs, degp, W, b, g, be):
    return pl.pallas_call(
        _tc_mid_body,
        grid=_GRID,
        in_specs=[_SPEC_ROWS, _SPEC_ROWS, _SPEC_DEGP, _SPEC_W,
                  _SPEC_VEC, _SPEC_VEC, _SPEC_VEC],
        out_specs=_SPEC_ROWS,
        out_shape=_OUT_ROWS,
    )(aggp, hs, degp, W, b.reshape(1, D), g.reshape(1, D), be.reshape(1, D))


def _tc_final(aggp, hs, degp, b, g, be):
    return pl.pallas_call(
        _tc_final_body,
        grid=_GRID,
        in_specs=[_SPEC_ROWS, _SPEC_ROWS, _SPEC_DEGP,
                  _SPEC_VEC, _SPEC_VEC, _SPEC_VEC],
        out_specs=_SPEC_ROWS,
        out_shape=_OUT_ROWS,
    )(aggp, hs, degp, b.reshape(1, D), g.reshape(1, D), be.reshape(1, D))


def kernel(x, edge_index, W1, b1, g1, be1, W2, b2, g2, be2, W3, b3, g3, be3):
    E = edge_index.shape[1]
    per = NW * CHUNK * 16
    E_pad = ((E + per - 1) // per) * per
    chunks_per_tile = E_pad // (NW * CHUNK)

    src = edge_index[0]
    dst = edge_index[1]
    pad = jnp.full((E_pad - E,), N_NODES, dtype=jnp.int32)
    src2d = jnp.concatenate([src, pad]).reshape(E_pad // 128, 128)
    dst2d = jnp.concatenate([dst, pad]).reshape(E_pad // 128, 128)

    x_pad = jnp.zeros((N_PAD, D), dtype=jnp.float32).at[:N_NODES].set(x)
    zrows = jnp.zeros((N_PAD, D), dtype=jnp.float32)
    ones_rows = jnp.ones((CHUNK, D), dtype=jnp.float32)

    deg_kernel = _make_deg_kernel(chunks_per_tile)
    agg_kernel = _make_agg_kernel(2 * chunks_per_tile)

    degp = deg_kernel(dst2d, ones_rows, zrows)

    hs1 = _tc_first(x_pad, degp, W1)
    agg1 = agg_kernel(hs1, src2d, dst2d, zrows)
    hs2 = _tc_mid(agg1, hs1, degp, W2, b1, g1, be1)
    agg2 = agg_kernel(hs2, src2d, dst2d, zrows)
    hs3 = _tc_mid(agg2, hs2, degp, W3, b2, g2, be2)
    agg3 = agg_kernel(hs3, src2d, dst2d, zrows)
    out = _tc_final(agg3, hs3, degp, b3, g3, be3)
    return out[:N_NODES]


# speedup vs baseline: 1.0042x; 1.0042x over previous
import functools
import math

import jax
import jax.numpy as jnp
from jax import lax
from jax.experimental import pallas as pl
from jax.experimental.pallas import tpu as pltpu
from jax.experimental.pallas import tpu_sc as plsc

N_NODES = 10000
D = 128
EPS = 1e-5
ISQ = 1.0 / math.sqrt(1.0 + EPS)

NC = 2
NS = 16
NW = NC * NS
CHUNK = 128
N_PAD = 10240

ROWS_PER_SUB = N_PAD // NS


def _sc_mesh():
    return plsc.VectorSubcoreMesh(
        core_axis_name="c", subcore_axis_name="s", num_cores=NC, num_subcores=NS
    )


def _make_deg_kernel(chunks_per_tile):
    @functools.partial(
        pl.kernel,
        mesh=_sc_mesh(),
        out_type=jax.ShapeDtypeStruct((NC, N_PAD, D), jnp.float32),
        scratch_types=[
            pltpu.VMEM((chunks_per_tile, CHUNK), jnp.int32),
            pltpu.VMEM((CHUNK, D), jnp.float32),
            pltpu.VMEM_SHARED((N_PAD, D), jnp.float32),
            pltpu.SemaphoreType.DMA,
        ],
    )
    def deg_kernel(dst_hbm, ones_hbm, z16_hbm, out_hbm, dst_v, ones_v, acc, sem):
        cid = lax.axis_index("c")
        sid = lax.axis_index("s")
        wid = sid * NC + cid
        pltpu.sync_copy(
            z16_hbm.at[pl.ds(sid * ROWS_PER_SUB, ROWS_PER_SUB)],
            acc.at[pl.ds(sid * ROWS_PER_SUB, ROWS_PER_SUB)],
        )
        pltpu.sync_copy(ones_hbm, ones_v)
        pltpu.sync_copy(
            dst_hbm.at[pl.ds(wid * chunks_per_tile, chunks_per_tile)], dst_v)
        plsc.subcore_barrier()

        def fire(j, carry):
            pltpu.async_copy(ones_v, acc.at[dst_v.at[j]], sem, add=True)
            return carry

        lax.fori_loop(0, chunks_per_tile, fire, 0)

        def drain(j, carry):
            pltpu.make_async_copy(ones_v, acc.at[dst_v.at[j]], sem).wait()
            return carry

        lax.fori_loop(0, chunks_per_tile, drain, 0)
        plsc.subcore_barrier()
        pltpu.sync_copy(
            acc.at[pl.ds(sid * ROWS_PER_SUB, ROWS_PER_SUB)],
            out_hbm.at[cid, pl.ds(sid * ROWS_PER_SUB, ROWS_PER_SUB)],
        )

    return deg_kernel


AGG_Q = 40


def _make_agg_kernel(n_chunks):
    assert n_chunks % AGG_Q == 0

    @functools.partial(
        pl.kernel,
        mesh=_sc_mesh(),
        out_type=jax.ShapeDtypeStruct((N_PAD, D), jnp.float32),
        scratch_types=[
            pltpu.VMEM((AGG_Q, CHUNK), jnp.int32),
            pltpu.VMEM((AGG_Q, CHUNK), jnp.int32),
            pltpu.VMEM((2 * CHUNK, D), jnp.float32),
            pltpu.VMEM_SHARED((N_PAD, D), jnp.float32),
            pltpu.SemaphoreType.DMA,
            pltpu.SemaphoreType.DMA,
        ],
    )
    def agg_kernel(hs_hbm, src_hbm, dst_hbm, z_hbm, out_hbm,
                   src_v, dst_v, rows_v, acc, gsem, ssem):
        cid = lax.axis_index("c")
        sid = lax.axis_index("s")

        @pl.when(cid == 0)
        def _():
            pltpu.sync_copy(
                z_hbm.at[pl.ds(sid * ROWS_PER_SUB, ROWS_PER_SUB)],
                acc.at[pl.ds(sid * ROWS_PER_SUB, ROWS_PER_SUB)],
            )
            plsc.subcore_barrier()

            q = AGG_Q
            for phase in range(n_chunks // q):
                base = sid * n_chunks + phase * q
                pltpu.sync_copy(src_hbm.at[pl.ds(base, q)], src_v)
                pltpu.sync_copy(dst_hbm.at[pl.ds(base, q)], dst_v)

                pltpu.async_copy(hs_hbm.at[src_v.at[0]],
                                 rows_v.at[pl.ds(0, CHUNK)], gsem)
                pltpu.async_copy(hs_hbm.at[src_v.at[1]],
                                 rows_v.at[pl.ds(CHUNK, CHUNK)], gsem)

                def body(j, carry):
                    b = lax.rem(j, 2) * CHUNK
                    bslc = rows_v.at[pl.ds(b, CHUNK)]
                    pltpu.make_async_copy(hs_hbm.at[src_v.at[j]], bslc,
                                          gsem).wait()
                    sdesc = pltpu.async_copy(bslc, acc.at[dst_v.at[j]], ssem,
                                             add=True)
                    sdesc.wait()

                    @pl.when(j + 2 < q)
                    def _():
                        pltpu.async_copy(hs_hbm.at[src_v.at[j + 2]], bslc, gsem)

                    return carry

                lax.fori_loop(0, q, body, 0)

            plsc.subcore_barrier()
            pltpu.sync_copy(
                acc.at[pl.ds(sid * ROWS_PER_SUB, ROWS_PER_SUB)],
                out_hbm.at[pl.ds(sid * ROWS_PER_SUB, ROWS_PER_SUB)],
            )

    return agg_kernel


BR = 1024


def _dis_of(degp_ref):
    return lax.rsqrt(degp_ref[0, :, 0:1] + degp_ref[1, :, 0:1] + 1.0)


def _tc_first_body(x_ref, degp_ref, w_ref, o_ref):
    dis = _dis_of(degp_ref)
    h = jnp.dot(x_ref[...], w_ref[...], preferred_element_type=jnp.float32)
    o_ref[...] = h * dis


def _tc_mid_body(agg_ref, hs_ref, degp_ref, w_ref, b_ref, g_ref, be_ref, o_ref):
    dis = _dis_of(degp_ref)
    conv = (agg_ref[...] + hs_ref[...]) * dis + b_ref[...]
    u = jnp.maximum(conv * (g_ref[...] * ISQ) + be_ref[...], 0.0)
    o_ref[...] = jnp.dot(u, w_ref[...], preferred_element_type=jnp.float32) * dis


def _tc_final_body(agg_ref, hs_ref, degp_ref, b_ref, g_ref, be_ref, o_ref):
    dis = _dis_of(degp_ref)
    conv = (agg_ref[...] + hs_ref[...]) * dis + b_ref[...]
    o_ref[...] = conv * (g_ref[...] * ISQ) + be_ref[...]


_GRID = (N_PAD // BR,)
_SPEC_ROWS = pl.BlockSpec((BR, D), lambda i: (i, 0))
_SPEC_DEGP = pl.BlockSpec((2, BR, D), lambda i: (0, i, 0))
_SPEC_W = pl.BlockSpec((D, D), lambda i: (0, 0))
_SPEC_VEC = pl.BlockSpec((1, D), lambda i: (0, 0))
_OUT_ROWS = jax.ShapeDtypeStruct((N_PAD, D), jnp.float32)


def _tc_first(x_pad, degp, W):
    return pl.pallas_call(
        _tc_first_body,
        grid=_GRID,
        in_specs=[_SPEC_ROWS, _SPEC_DEGP, _SPEC_W],
        out_specs=_SPEC_ROWS,
        out_shape=_OUT_ROWS,
    )(x_pad, degp, W)


def _tc_mid(aggp, hs, degp, W, b, g, be):
    return pl.pallas_call(
        _tc_mid_body,
        grid=_GRID,
        in_specs=[_SPEC_ROWS, _SPEC_ROWS, _SPEC_DEGP, _SPEC_W,
                  _SPEC_VEC, _SPEC_VEC, _SPEC_VEC],
        out_specs=_SPEC_ROWS,
        out_shape=_OUT_ROWS,
    )(aggp, hs, degp, W, b.reshape(1, D), g.reshape(1, D), be.reshape(1, D))


def _tc_final(aggp, hs, degp, b, g, be):
    return pl.pallas_call(
        _tc_final_body,
        grid=_GRID,
        in_specs=[_SPEC_ROWS, _SPEC_ROWS, _SPEC_DEGP,
                  _SPEC_VEC, _SPEC_VEC, _SPEC_VEC],
        out_specs=_SPEC_ROWS,
        out_shape=_OUT_ROWS,
    )(aggp, hs, degp, b.reshape(1, D), g.reshape(1, D), be.reshape(1, D))


def kernel(x, edge_index, W1, b1, g1, be1, W2, b2, g2, be2, W3, b3, g3, be3):
    E = edge_index.shape[1]
    per = NW * CHUNK * 16
    E_pad = ((E + per - 1) // per) * per
    chunks_per_tile = E_pad // (NW * CHUNK)

    src = edge_index[0]
    dst = edge_index[1]
    pad = jnp.full((E_pad - E,), N_NODES, dtype=jnp.int32)
    src2d = jnp.concatenate([src, pad]).reshape(E_pad // 128, 128)
    dst2d = jnp.concatenate([dst, pad]).reshape(E_pad // 128, 128)

    x_pad = jnp.zeros((N_PAD, D), dtype=jnp.float32).at[:N_NODES].set(x)
    zrows = jnp.zeros((N_PAD, D), dtype=jnp.float32)
    ones_rows = jnp.ones((CHUNK, D), dtype=jnp.float32)

    deg_kernel = _make_deg_kernel(chunks_per_tile)
    agg_kernel = _make_agg_kernel(2 * chunks_per_tile)

    degp = deg_kernel(dst2d, ones_rows, zrows)

    hs1 = _tc_first(x_pad, degp, W1)
    agg1 = agg_kernel(hs1, src2d, dst2d, zrows)
    hs2 = _tc_mid(agg1, hs1, degp, W2, b1, g1, be1)
    agg2 = agg_kernel(hs2, src2d, dst2d, zrows)
    hs3 = _tc_mid(agg2, hs2, degp, W3, b2, g2, be2)
    agg3 = agg_kernel(hs3, src2d, dst2d, zrows)
    out = _tc_final(agg3, hs3, degp, b3, g3, be3)
    return out[:N_NODES]


# speedup vs baseline: 2.6883x; 2.6771x over previous
import functools
import math

import jax
import jax.numpy as jnp
from jax import lax
from jax.experimental import pallas as pl
from jax.experimental.pallas import tpu as pltpu
from jax.experimental.pallas import tpu_sc as plsc

N_NODES = 10000
D = 128
EPS = 1e-5
ISQ = 1.0 / math.sqrt(1.0 + EPS)

NC = 2
NS = 16
NW = NC * NS
CHUNK = 128
N_PAD = 10240

ROWS_PER_SUB = N_PAD // NS


def _sc_mesh():
    return plsc.VectorSubcoreMesh(
        core_axis_name="c", subcore_axis_name="s", num_cores=NC, num_subcores=NS
    )


def _make_deg_kernel(chunks_per_tile):
    @functools.partial(
        pl.kernel,
        mesh=_sc_mesh(),
        out_type=jax.ShapeDtypeStruct((NC, N_PAD, D), jnp.float32),
        scratch_types=[
            pltpu.VMEM((chunks_per_tile, CHUNK), jnp.int32),
            pltpu.VMEM((CHUNK, D), jnp.float32),
            pltpu.VMEM_SHARED((N_PAD, D), jnp.float32),
            pltpu.SemaphoreType.DMA,
        ],
    )
    def deg_kernel(dst_hbm, ones_hbm, z16_hbm, out_hbm, dst_v, ones_v, acc, sem):
        cid = lax.axis_index("c")
        sid = lax.axis_index("s")
        wid = sid * NC + cid
        pltpu.sync_copy(
            z16_hbm.at[pl.ds(sid * ROWS_PER_SUB, ROWS_PER_SUB)],
            acc.at[pl.ds(sid * ROWS_PER_SUB, ROWS_PER_SUB)],
        )
        pltpu.sync_copy(ones_hbm, ones_v)
        pltpu.sync_copy(
            dst_hbm.at[pl.ds(wid * chunks_per_tile, chunks_per_tile)], dst_v)
        plsc.subcore_barrier()

        def fire(j, carry):
            pltpu.async_copy(ones_v, acc.at[dst_v.at[j]], sem, add=True)
            return carry

        lax.fori_loop(0, chunks_per_tile, fire, 0)

        def drain(j, carry):
            pltpu.make_async_copy(ones_v, acc.at[dst_v.at[j]], sem).wait()
            return carry

        lax.fori_loop(0, chunks_per_tile, drain, 0)
        plsc.subcore_barrier()
        pltpu.sync_copy(
            acc.at[pl.ds(sid * ROWS_PER_SUB, ROWS_PER_SUB)],
            out_hbm.at[cid, pl.ds(sid * ROWS_PER_SUB, ROWS_PER_SUB)],
        )

    return deg_kernel


AGG_Q = 40


def _make_agg_kernel(n_chunks):
    assert n_chunks % AGG_Q == 0

    @functools.partial(
        pl.kernel,
        mesh=_sc_mesh(),
        out_type=jax.ShapeDtypeStruct((N_PAD, D), jnp.float32),
        scratch_types=[
            pltpu.VMEM((AGG_Q, CHUNK), jnp.int32),
            pltpu.VMEM((AGG_Q, CHUNK), jnp.int32),
            pltpu.VMEM((2 * CHUNK, D), jnp.float32),
            pltpu.VMEM_SHARED((N_PAD, D), jnp.float32),
            pltpu.SemaphoreType.DMA,
            pltpu.SemaphoreType.DMA,
        ],
    )
    def agg_kernel(hs_hbm, src_hbm, dst_hbm, z_hbm, out_hbm,
                   src_v, dst_v, rows_v, acc, gsem, ssem):
        cid = lax.axis_index("c")
        sid = lax.axis_index("s")

        @pl.when(cid == 0)
        def _():
            pltpu.sync_copy(
                z_hbm.at[pl.ds(sid * ROWS_PER_SUB, ROWS_PER_SUB)],
                acc.at[pl.ds(sid * ROWS_PER_SUB, ROWS_PER_SUB)],
            )
            plsc.subcore_barrier()

            q = AGG_Q
            for phase in range(n_chunks // q):
                base = sid * n_chunks + phase * q
                pltpu.sync_copy(src_hbm.at[pl.ds(base, q)], src_v)
                pltpu.sync_copy(dst_hbm.at[pl.ds(base, q)], dst_v)

                pltpu.async_copy(hs_hbm.at[src_v.at[0]],
                                 rows_v.at[pl.ds(0, CHUNK)], gsem)
                pltpu.async_copy(hs_hbm.at[src_v.at[1]],
                                 rows_v.at[pl.ds(CHUNK, CHUNK)], gsem)

                def body(j, carry):
                    b = lax.rem(j, 2) * CHUNK
                    bslc = rows_v.at[pl.ds(b, CHUNK)]
                    pltpu.make_async_copy(hs_hbm.at[src_v.at[j]], bslc,
                                          gsem).wait()
                    sdesc = pltpu.async_copy(bslc, acc.at[dst_v.at[j]], ssem,
                                             add=True)
                    sdesc.wait()

                    @pl.when(j + 2 < q)
                    def _():
                        pltpu.async_copy(hs_hbm.at[src_v.at[j + 2]], bslc, gsem)

                    return carry

                lax.fori_loop(0, q, body, 0)

            plsc.subcore_barrier()
            pltpu.sync_copy(
                acc.at[pl.ds(sid * ROWS_PER_SUB, ROWS_PER_SUB)],
                out_hbm.at[pl.ds(sid * ROWS_PER_SUB, ROWS_PER_SUB)],
            )

    return agg_kernel


BR = 1024


def _dis_of(degp_ref):
    return lax.rsqrt(degp_ref[0, :, 0:1] + degp_ref[1, :, 0:1] + 1.0)


def _tc_first_body(x_ref, degp_ref, w_ref, o_ref):
    dis = _dis_of(degp_ref)
    h = jnp.dot(x_ref[...], w_ref[...], preferred_element_type=jnp.float32)
    o_ref[...] = h * dis


def _tc_mid_body(agg_ref, hs_ref, degp_ref, w_ref, b_ref, g_ref, be_ref, o_ref):
    dis = _dis_of(degp_ref)
    conv = (agg_ref[...] + hs_ref[...]) * dis + b_ref[...]
    u = jnp.maximum(conv * (g_ref[...] * ISQ) + be_ref[...], 0.0)
    o_ref[...] = jnp.dot(u, w_ref[...], preferred_element_type=jnp.float32) * dis


def _tc_final_body(agg_ref, hs_ref, degp_ref, b_ref, g_ref, be_ref, o_ref):
    dis = _dis_of(degp_ref)
    conv = (agg_ref[...] + hs_ref[...]) * dis + b_ref[...]
    o_ref[...] = conv * (g_ref[...] * ISQ) + be_ref[...]


_GRID = (N_PAD // BR,)
_SPEC_ROWS = pl.BlockSpec((BR, D), lambda i: (i, 0))
_SPEC_DEGP = pl.BlockSpec((2, BR, D), lambda i: (0, i, 0))
_SPEC_W = pl.BlockSpec((D, D), lambda i: (0, 0))
_SPEC_VEC = pl.BlockSpec((1, D), lambda i: (0, 0))
_OUT_ROWS = jax.ShapeDtypeStruct((N_PAD, D), jnp.float32)


def _tc_first(x_pad, degp, W):
    return pl.pallas_call(
        _tc_first_body,
        grid=_GRID,
        in_specs=[_SPEC_ROWS, _SPEC_DEGP, _SPEC_W],
        out_specs=_SPEC_ROWS,
        out_shape=_OUT_ROWS,
    )(x_pad, degp, W)


def _tc_mid(aggp, hs, degp, W, b, g, be):
    return pl.pallas_call(
        _tc_mid_body,
        grid=_GRID,
        in_specs=[_SPEC_ROWS, _SPEC_ROWS, _SPEC_DEGP, _SPEC_W,
                  _SPEC_VEC, _SPEC_VEC, _SPEC_VEC],
        out_specs=_SPEC_ROWS,
        out_shape=_OUT_ROWS,
    )(aggp, hs, degp, W, b.reshape(1, D), g.reshape(1, D), be.reshape(1, D))


def _tc_final(aggp, hs, degp, b, g, be):
    return pl.pallas_call(
        _tc_final_body,
        grid=_GRID,
        in_specs=[_SPEC_ROWS, _SPEC_ROWS, _SPEC_DEGP,
                  _SPEC_VEC, _SPEC_VEC, _SPEC_VEC],
        out_specs=_SPEC_ROWS,
        out_shape=_OUT_ROWS,
    )(aggp, hs, degp, b.reshape(1, D), g.reshape(1, D), be.reshape(1, D))


def kernel(x, edge_index, W1, b1, g1, be1, W2, b2, g2, be2, W3, b3, g3, be3):
    E = edge_index.shape[1]
    per = NW * CHUNK * 16
    E_pad = ((E + per - 1) // per) * per
    chunks_per_tile = E_pad // (NW * CHUNK)

    src = edge_index[0]
    dst = edge_index[1]
    pad = N_NODES + (jnp.arange(E_pad - E, dtype=jnp.int32) % (N_PAD - N_NODES))
    src2d = jnp.concatenate([src, pad]).reshape(E_pad // 128, 128)
    dst2d = jnp.concatenate([dst, pad]).reshape(E_pad // 128, 128)

    x_pad = jnp.zeros((N_PAD, D), dtype=jnp.float32).at[:N_NODES].set(x)
    zrows = jnp.zeros((N_PAD, D), dtype=jnp.float32)
    ones_rows = jnp.ones((CHUNK, D), dtype=jnp.float32)

    deg_kernel = _make_deg_kernel(chunks_per_tile)
    agg_kernel = _make_agg_kernel(2 * chunks_per_tile)

    degp = deg_kernel(dst2d, ones_rows, zrows)

    hs1 = _tc_first(x_pad, degp, W1)
    agg1 = agg_kernel(hs1, src2d, dst2d, zrows)
    hs2 = _tc_mid(agg1, hs1, degp, W2, b1, g1, be1)
    agg2 = agg_kernel(hs2, src2d, dst2d, zrows)
    hs3 = _tc_mid(agg2, hs2, degp, W3, b2, g2, be2)
    agg3 = agg_kernel(hs3, src2d, dst2d, zrows)
    out = _tc_final(agg3, hs3, degp, b3, g3, be3)
    return out[:N_NODES]


# speedup vs baseline: 4.1310x; 1.5367x over previous
import functools
import math

import jax
import jax.numpy as jnp
from jax import lax
from jax.experimental import pallas as pl
from jax.experimental.pallas import tpu as pltpu
from jax.experimental.pallas import tpu_sc as plsc

N_NODES = 10000
D = 128
EPS = 1e-5
ISQ = 1.0 / math.sqrt(1.0 + EPS)

NC = 2
NS = 16
NW = NC * NS
CHUNK = 128
N_PAD = 10240

ROWS_PER_SUB = N_PAD // NS


def _sc_mesh():
    return plsc.VectorSubcoreMesh(
        core_axis_name="c", subcore_axis_name="s", num_cores=NC, num_subcores=NS
    )


def _make_deg_kernel(chunks_per_tile):
    @functools.partial(
        pl.kernel,
        mesh=_sc_mesh(),
        out_type=jax.ShapeDtypeStruct((NC, N_PAD, D), jnp.float32),
        scratch_types=[
            pltpu.VMEM((chunks_per_tile, CHUNK), jnp.int32),
            pltpu.VMEM((CHUNK, D), jnp.float32),
            pltpu.VMEM_SHARED((N_PAD, D), jnp.float32),
            pltpu.SemaphoreType.DMA,
        ],
    )
    def deg_kernel(dst_hbm, ones_hbm, z16_hbm, out_hbm, dst_v, ones_v, acc, sem):
        cid = lax.axis_index("c")
        sid = lax.axis_index("s")
        wid = sid * NC + cid
        pltpu.sync_copy(
            z16_hbm.at[pl.ds(sid * ROWS_PER_SUB, ROWS_PER_SUB)],
            acc.at[pl.ds(sid * ROWS_PER_SUB, ROWS_PER_SUB)],
        )
        pltpu.sync_copy(ones_hbm, ones_v)
        pltpu.sync_copy(
            dst_hbm.at[pl.ds(wid * chunks_per_tile, chunks_per_tile)], dst_v)
        plsc.subcore_barrier()

        def fire(j, carry):
            pltpu.async_copy(ones_v, acc.at[dst_v.at[j]], sem, add=True)
            return carry

        lax.fori_loop(0, chunks_per_tile, fire, 0)

        def drain(j, carry):
            pltpu.make_async_copy(ones_v, acc.at[dst_v.at[j]], sem).wait()
            return carry

        lax.fori_loop(0, chunks_per_tile, drain, 0)
        plsc.subcore_barrier()
        pltpu.sync_copy(
            acc.at[pl.ds(sid * ROWS_PER_SUB, ROWS_PER_SUB)],
            out_hbm.at[cid, pl.ds(sid * ROWS_PER_SUB, ROWS_PER_SUB)],
        )

    return deg_kernel


AGG_Q = 40


def _make_agg_kernel(n_chunks):
    assert n_chunks % AGG_Q == 0

    @functools.partial(
        pl.kernel,
        mesh=_sc_mesh(),
        out_type=jax.ShapeDtypeStruct((NC, N_PAD, D), jnp.float32),
        scratch_types=[
            pltpu.VMEM((AGG_Q, CHUNK), jnp.int32),
            pltpu.VMEM((AGG_Q, CHUNK), jnp.int32),
            pltpu.VMEM((2 * CHUNK, D), jnp.float32),
            pltpu.VMEM_SHARED((N_PAD, D), jnp.float32),
            pltpu.SemaphoreType.DMA,
            pltpu.SemaphoreType.DMA,
        ],
    )
    def agg_kernel(hs_hbm, src_hbm, dst_hbm, z_hbm, out_hbm,
                   src_v, dst_v, rows_v, acc, gsem, ssem):
        cid = lax.axis_index("c")
        sid = lax.axis_index("s")
        wid = cid * NS + sid
        pltpu.sync_copy(
            z_hbm.at[pl.ds(sid * ROWS_PER_SUB, ROWS_PER_SUB)],
            acc.at[pl.ds(sid * ROWS_PER_SUB, ROWS_PER_SUB)],
        )
        plsc.subcore_barrier()

        q = AGG_Q
        for phase in range(n_chunks // q):
            base = wid * n_chunks + phase * q
            pltpu.sync_copy(src_hbm.at[pl.ds(base, q)], src_v)
            pltpu.sync_copy(dst_hbm.at[pl.ds(base, q)], dst_v)

            pltpu.async_copy(hs_hbm.at[src_v.at[0]],
                             rows_v.at[pl.ds(0, CHUNK)], gsem)
            pltpu.async_copy(hs_hbm.at[src_v.at[1]],
                             rows_v.at[pl.ds(CHUNK, CHUNK)], gsem)

            def body(j, carry):
                b = lax.rem(j, 2) * CHUNK
                bslc = rows_v.at[pl.ds(b, CHUNK)]
                pltpu.make_async_copy(hs_hbm.at[src_v.at[j]], bslc,
                                      gsem).wait()
                sdesc = pltpu.async_copy(bslc, acc.at[dst_v.at[j]], ssem,
                                         add=True)
                sdesc.wait()

                @pl.when(j + 2 < q)
                def _():
                    pltpu.async_copy(hs_hbm.at[src_v.at[j + 2]], bslc, gsem)

                return carry

            lax.fori_loop(0, q, body, 0)

        plsc.subcore_barrier()
        pltpu.sync_copy(
            acc.at[pl.ds(sid * ROWS_PER_SUB, ROWS_PER_SUB)],
            out_hbm.at[cid, pl.ds(sid * ROWS_PER_SUB, ROWS_PER_SUB)],
        )

    return agg_kernel


BR = 1024


def _dis_of(degp_ref):
    return lax.rsqrt(degp_ref[0, :, 0:1] + degp_ref[1, :, 0:1] + 1.0)


def _tc_first_body(x_ref, degp_ref, w_ref, o_ref):
    dis = _dis_of(degp_ref)
    h = jnp.dot(x_ref[...], w_ref[...], preferred_element_type=jnp.float32)
    o_ref[...] = h * dis


def _tc_mid_body(agg_ref, hs_ref, degp_ref, w_ref, b_ref, g_ref, be_ref, o_ref):
    dis = _dis_of(degp_ref)
    conv = (agg_ref[0] + agg_ref[1] + hs_ref[...]) * dis + b_ref[...]
    u = jnp.maximum(conv * (g_ref[...] * ISQ) + be_ref[...], 0.0)
    o_ref[...] = jnp.dot(u, w_ref[...], preferred_element_type=jnp.float32) * dis


def _tc_final_body(agg_ref, hs_ref, degp_ref, b_ref, g_ref, be_ref, o_ref):
    dis = _dis_of(degp_ref)
    conv = (agg_ref[0] + agg_ref[1] + hs_ref[...]) * dis + b_ref[...]
    o_ref[...] = conv * (g_ref[...] * ISQ) + be_ref[...]


_GRID = (N_PAD // BR,)
_SPEC_ROWS = pl.BlockSpec((BR, D), lambda i: (i, 0))
_SPEC_DEGP = pl.BlockSpec((2, BR, D), lambda i: (0, i, 0))
_SPEC_W = pl.BlockSpec((D, D), lambda i: (0, 0))
_SPEC_VEC = pl.BlockSpec((1, D), lambda i: (0, 0))
_OUT_ROWS = jax.ShapeDtypeStruct((N_PAD, D), jnp.float32)


def _tc_first(x_pad, degp, W):
    return pl.pallas_call(
        _tc_first_body,
        grid=_GRID,
        in_specs=[_SPEC_ROWS, _SPEC_DEGP, _SPEC_W],
        out_specs=_SPEC_ROWS,
        out_shape=_OUT_ROWS,
    )(x_pad, degp, W)


def _tc_mid(aggp, hs, degp, W, b, g, be):
    return pl.pallas_call(
        _tc_mid_body,
        grid=_GRID,
        in_specs=[_SPEC_DEGP, _SPEC_ROWS, _SPEC_DEGP, _SPEC_W,
                  _SPEC_VEC, _SPEC_VEC, _SPEC_VEC],
        out_specs=_SPEC_ROWS,
        out_shape=_OUT_ROWS,
    )(aggp, hs, degp, W, b.reshape(1, D), g.reshape(1, D), be.reshape(1, D))


def _tc_final(aggp, hs, degp, b, g, be):
    return pl.pallas_call(
        _tc_final_body,
        grid=_GRID,
        in_specs=[_SPEC_DEGP, _SPEC_ROWS, _SPEC_DEGP,
                  _SPEC_VEC, _SPEC_VEC, _SPEC_VEC],
        out_specs=_SPEC_ROWS,
        out_shape=_OUT_ROWS,
    )(aggp, hs, degp, b.reshape(1, D), g.reshape(1, D), be.reshape(1, D))


def kernel(x, edge_index, W1, b1, g1, be1, W2, b2, g2, be2, W3, b3, g3, be3):
    E = edge_index.shape[1]
    per = NW * CHUNK * 16
    E_pad = ((E + per - 1) // per) * per
    chunks_per_tile = E_pad // (NW * CHUNK)

    src = edge_index[0]
    dst = edge_index[1]
    pad = N_NODES + (jnp.arange(E_pad - E, dtype=jnp.int32) % (N_PAD - N_NODES))
    src2d = jnp.concatenate([src, pad]).reshape(E_pad // 128, 128)
    dst2d = jnp.concatenate([dst, pad]).reshape(E_pad // 128, 128)

    x_pad = jnp.zeros((N_PAD, D), dtype=jnp.float32).at[:N_NODES].set(x)
    zrows = jnp.zeros((N_PAD, D), dtype=jnp.float32)
    ones_rows = jnp.ones((CHUNK, D), dtype=jnp.float32)

    deg_kernel = _make_deg_kernel(chunks_per_tile)
    agg_kernel = _make_agg_kernel(chunks_per_tile)

    degp = deg_kernel(dst2d, ones_rows, zrows)

    hs1 = _tc_first(x_pad, degp, W1)
    agg1 = agg_kernel(hs1, src2d, dst2d, zrows)
    hs2 = _tc_mid(agg1, hs1, degp, W2, b1, g1, be1)
    agg2 = agg_kernel(hs2, src2d, dst2d, zrows)
    hs3 = _tc_mid(agg2, hs2, degp, W3, b2, g2, be2)
    agg3 = agg_kernel(hs3, src2d, dst2d, zrows)
    out = _tc_final(agg3, hs3, degp, b3, g3, be3)
    return out[:N_NODES]
